# Initial kernel scaffold; baseline (speedup 1.0000x reference)
import jax
import jax.numpy as jnp
from jax.experimental import pallas as pl


def kernel(means, covs, weights, seed):
    raise NotImplementedError("write your pallas kernel here")


# speedup vs baseline: 5.7219x; 5.7219x over previous
import functools

import jax
import jax.numpy as jnp
import numpy as np
from jax import lax
from jax.experimental import pallas as pl
from jax.experimental.pallas import tpu as pltpu
from jax.experimental.pallas import tpu_sc as plsc

NG = 4096
MC = 64
D = 16

_I32 = jnp.int32
_KS_PARITY = 0x1BD11BDA

_LO = np.float32(np.nextafter(np.float32(-1.0), np.float32(0.0)))
_HI = np.float32(1.0)
_SPAN = np.float32(_HI - _LO)
_SQRT2 = np.float32(np.sqrt(2.0))


def _lsr(x, s):
    return (x >> s) & _I32((1 << (32 - s)) - 1)


def _rotl(x, r):
    return (x << r) | _lsr(x, 32 - r)


def _threefry2x32(k0, k1, x0, x1):
    ks2 = k0 ^ k1 ^ _I32(_KS_PARITY)
    ks = (k0, k1, ks2)
    rots = ((13, 15, 26, 6), (17, 29, 16, 24))
    x0 = x0 + k0
    x1 = x1 + k1
    for i in range(5):
        for r in rots[i % 2]:
            x0 = x0 + x1
            x1 = _rotl(x1, r)
            x1 = x0 ^ x1
        x0 = x0 + ks[(i + 1) % 3]
        x1 = x1 + ks[(i + 2) % 3] + _I32(i + 1)
    return x0, x1


def _bits_to_unit(bits):
    fb = _lsr(bits, 9) | _I32(0x3F800000)
    return lax.bitcast_convert_type(fb, jnp.float32) - jnp.float32(1.0)


def _erf_inv(x):
    w = -jnp.log1p(-x * x)
    ws = w - jnp.float32(2.5)
    p = jnp.float32(2.81022636e-08)
    for c in (3.43273939e-07, -3.5233877e-06, -4.39150654e-06, 0.00021858087,
              -0.00125372503, -0.00417768164, 0.246640727, 1.50140941):
        p = jnp.float32(c) + p * ws
    wb = jnp.sqrt(w) - jnp.float32(3.0)
    q = jnp.float32(-0.000200214257)
    for c in (0.000100950558, 0.00134934322, -0.00367342844, 0.00573950773,
              -0.0076224613, 0.00943887047, 1.00167406, 2.83297682):
        q = jnp.float32(c) + q * wb
    return jnp.where(w < jnp.float32(5.0), p, q) * x


def _prep_body(wt_ref, seed_ref, gidx_ref, z_ref, oh_ref):
    seed = seed_ref[0, 0]
    lane = lax.broadcasted_iota(_I32, (1, NG), 1)

    ka, kb = _threefry2x32(_I32(0) * lane, _I32(0) * lane + seed,
                           _I32(0) * lane, lane)
    ca, cb = _threefry2x32(ka, kb, _I32(0) * lane, _I32(0) * lane)
    sa, sb = _threefry2x32(ka, kb, _I32(0) * lane, _I32(0) * lane + 1)

    ua, ub = _threefry2x32(ca, cb, _I32(0) * lane, _I32(0) * lane)
    u = _bits_to_unit(ua ^ ub)

    rows = [wt_ref[0:1, :]]
    for j in range(1, MC):
        rows.append(rows[-1] + wt_ref[j:j + 1, :])
    r = rows[-1] * (jnp.float32(1.0) - u)

    cnt = _I32(0) * lane
    for j in range(MC):
        cnt = cnt + jnp.where(rows[j] < r, _I32(1), _I32(0))
    gidx_ref[:, :] = lane * MC + cnt
    low = cnt & 7
    for o in range(8):
        oh_ref[o:o + 1, :] = jnp.where(low == o, jnp.float32(1.0),
                                       jnp.float32(0.0))

    saf = jnp.broadcast_to(sa, (D, NG))
    sbf = jnp.broadcast_to(sb, (D, NG))
    cj = lax.broadcasted_iota(_I32, (D, NG), 0)
    za, zb = _threefry2x32(saf, sbf, _I32(0) * cj, cj)
    zu = _bits_to_unit(za ^ zb)
    u2 = jnp.maximum(_LO, zu * _SPAN + _LO)
    z_ref[:, :] = _SQRT2 * _erf_inv(u2)


def _chol_body(cov_ref, mrow_ref, oh_ref, z_ref, out_ref):
    R = cov_ref.shape[0]
    ilane = lax.broadcasted_iota(_I32, (R, D), 1)
    out = mrow_ref[:, 0:D] * oh_ref[:, 0:1]
    for o in range(1, 8):
        out = out + mrow_ref[:, D * o:D * (o + 1)] * oh_ref[:, o:o + 1]
    cols = []
    for j in range(D):
        acc = cov_ref[:, D * j:D * (j + 1)]
        for k in range(j):
            acc = acc - cols[k] * cols[k][:, j:j + 1]
        sq = jnp.sqrt(acc[:, j:j + 1])
        colj = jnp.where(ilane >= j, acc / sq, jnp.float32(0.0))
        out = out + colj * z_ref[:, j:j + 1]
        cols.append(colj)
    out_ref[:, :] = out


def _sc_gather(covs2, means8, gidx):
    info = plsc.get_sparse_core_info()
    nw = info.num_cores * info.num_subcores
    nb = NG // nw
    mesh = plsc.VectorSubcoreMesh(core_axis_name="c", subcore_axis_name="s")

    @functools.partial(
        pl.kernel,
        mesh=mesh,
        out_type=[
            jax.ShapeDtypeStruct((NG, D * D), jnp.float32),
            jax.ShapeDtypeStruct((NG, 128), jnp.float32),
        ],
        scratch_types=[
            pltpu.VMEM((nb,), jnp.int32),
            pltpu.VMEM((nb,), jnp.int32),
            pltpu.VMEM((nb, D * D), jnp.float32),
            pltpu.VMEM((nb, 128), jnp.float32),
            pltpu.SemaphoreType.DMA,
            pltpu.SemaphoreType.DMA,
        ],
    )
    def gather_k(covs_hbm, means_hbm, idx_hbm, cov_out, mrow_out,
                 idx_v, idx8_v, cov_v, mrow_v, sem1, sem2):
        wid = lax.axis_index("s") * info.num_cores + lax.axis_index("c")
        base = wid * nb
        pltpu.sync_copy(idx_hbm.at[pl.ds(base, nb)], idx_v)
        for t in range(nb // 16):
            idx8_v[pl.ds(16 * t, 16)] = idx_v[pl.ds(16 * t, 16)] >> 3
        c1 = pltpu.async_copy(covs_hbm.at[idx_v], cov_v, sem1)
        c2 = pltpu.async_copy(means_hbm.at[idx8_v], mrow_v, sem2)
        c2.wait()
        pltpu.sync_copy(mrow_v, mrow_out.at[pl.ds(base, nb)])
        c1.wait()
        pltpu.sync_copy(cov_v, cov_out.at[pl.ds(base, nb)])

    return gather_k(covs2, means8, gidx)


_prep_call = pl.pallas_call(
    _prep_body,
    out_shape=[
        jax.ShapeDtypeStruct((1, NG), jnp.int32),
        jax.ShapeDtypeStruct((D, NG), jnp.float32),
        jax.ShapeDtypeStruct((8, NG), jnp.float32),
    ],
    in_specs=[
        pl.BlockSpec(memory_space=pltpu.VMEM),
        pl.BlockSpec(memory_space=pltpu.SMEM),
    ],
    out_specs=[
        pl.BlockSpec(memory_space=pltpu.VMEM),
        pl.BlockSpec(memory_space=pltpu.VMEM),
        pl.BlockSpec(memory_space=pltpu.VMEM),
    ],
)

_CHOL_R = 512

_chol_call = pl.pallas_call(
    _chol_body,
    grid=(NG // _CHOL_R,),
    in_specs=[
        pl.BlockSpec((_CHOL_R, D * D), lambda i: (i, 0)),
        pl.BlockSpec((_CHOL_R, 128), lambda i: (i, 0)),
        pl.BlockSpec((_CHOL_R, 8), lambda i: (i, 0)),
        pl.BlockSpec((_CHOL_R, D), lambda i: (i, 0)),
    ],
    out_specs=pl.BlockSpec((_CHOL_R, D), lambda i: (i, 0)),
    out_shape=jax.ShapeDtypeStruct((NG, D), jnp.float32),
)


def kernel(means, covs, weights, seed):
    wt = weights.T
    seed_arr = jnp.asarray(seed, jnp.int32).reshape(1, 1)
    gidx2, z_t, oh = _prep_call(wt, seed_arr)
    gidx = gidx2.reshape(NG)
    z = z_t.T
    oh_t = oh.T
    covs2 = covs.reshape(NG * MC, D * D)
    means8 = means.reshape(NG * MC * D // 128, 128)
    cov_rows, mrows = _sc_gather(covs2, means8, gidx)
    return _chol_call(cov_rows, mrows, oh_t, z)


# speedup vs baseline: 61.8342x; 10.8066x over previous
import jax
import jax.numpy as jnp
import numpy as np
from jax import lax
from jax.experimental import pallas as pl
from jax.experimental.pallas import tpu as pltpu

NG = 4096
MC = 64
D = 16

_I32 = jnp.int32
_KS_PARITY = 0x1BD11BDA

_LO = np.float32(np.nextafter(np.float32(-1.0), np.float32(0.0)))
_HI = np.float32(1.0)
_SPAN = np.float32(_HI - _LO)
_SQRT2 = np.float32(np.sqrt(2.0))


def _lsr(x, s):
    return (x >> s) & _I32((1 << (32 - s)) - 1)


def _rotl(x, r):
    return (x << r) | _lsr(x, 32 - r)


def _threefry2x32(k0, k1, x0, x1):
    ks2 = k0 ^ k1 ^ _I32(_KS_PARITY)
    ks = (k0, k1, ks2)
    rots = ((13, 15, 26, 6), (17, 29, 16, 24))
    x0 = x0 + k0
    x1 = x1 + k1
    for i in range(5):
        for r in rots[i % 2]:
            x0 = x0 + x1
            x1 = _rotl(x1, r)
            x1 = x0 ^ x1
        x0 = x0 + ks[(i + 1) % 3]
        x1 = x1 + ks[(i + 2) % 3] + _I32(i + 1)
    return x0, x1


def _bits_to_unit(bits):
    fb = _lsr(bits, 9) | _I32(0x3F800000)
    return lax.bitcast_convert_type(fb, jnp.float32) - jnp.float32(1.0)


def _erf_inv(x):
    w = -jnp.log1p(-x * x)
    ws = w - jnp.float32(2.5)
    p = jnp.float32(2.81022636e-08)
    for c in (3.43273939e-07, -3.5233877e-06, -4.39150654e-06, 0.00021858087,
              -0.00125372503, -0.00417768164, 0.246640727, 1.50140941):
        p = jnp.float32(c) + p * ws
    wb = jnp.sqrt(w) - jnp.float32(3.0)
    q = jnp.float32(-0.000200214257)
    for c in (0.000100950558, 0.00134934322, -0.00367342844, 0.00573950773,
              -0.0076224613, 0.00943887047, 1.00167406, 2.83297682):
        q = jnp.float32(c) + q * wb
    return jnp.where(w < jnp.float32(5.0), p, q) * x


def _prep_body(wt_ref, seed_ref, comp_ref, z_ref):
    seed = seed_ref[0, 0]
    lane = lax.broadcasted_iota(_I32, (1, NG), 1)

    ka, kb = _threefry2x32(_I32(0) * lane, _I32(0) * lane + seed,
                           _I32(0) * lane, lane)
    ca, cb = _threefry2x32(ka, kb, _I32(0) * lane, _I32(0) * lane)
    sa, sb = _threefry2x32(ka, kb, _I32(0) * lane, _I32(0) * lane + 1)

    ua, ub = _threefry2x32(ca, cb, _I32(0) * lane, _I32(0) * lane)
    u = _bits_to_unit(ua ^ ub)

    rows = [wt_ref[0:1, :]]
    for j in range(1, MC):
        rows.append(rows[-1] + wt_ref[j:j + 1, :])
    r = rows[-1] * (jnp.float32(1.0) - u)

    cnt = _I32(0) * lane
    for j in range(MC):
        cnt = cnt + jnp.where(rows[j] < r, _I32(1), _I32(0))
    comp_ref[:, :] = cnt

    saf = jnp.broadcast_to(sa, (D, NG))
    sbf = jnp.broadcast_to(sb, (D, NG))
    cj = lax.broadcasted_iota(_I32, (D, NG), 0)
    za, zb = _threefry2x32(saf, sbf, _I32(0) * cj, cj)
    zu = _bits_to_unit(za ^ zb)
    u2 = jnp.maximum(_LO, zu * _SPAN + _LO)
    z_ref[:, :] = _SQRT2 * _erf_inv(u2)


def _sweep_body(comp_ref, z_ref, covt_ref, meant_ref, out_ref,
                selcov, selmean):
    c = pl.program_id(0)
    mask = comp_ref[:, :] == c
    m3 = mask.reshape(1, 1, NG)

    @pl.when(c == 0)
    def _():
        selcov[...] = jnp.where(m3, covt_ref[0], jnp.float32(0.0))
        selmean[...] = jnp.where(mask, meant_ref[0], jnp.float32(0.0))

    @pl.when(c > 0)
    def _():
        selcov[...] = jnp.where(m3, covt_ref[0], selcov[...])
        selmean[...] = jnp.where(mask, meant_ref[0], selmean[...])

    @pl.when(c == MC - 1)
    def _():
        isub = lax.broadcasted_iota(_I32, (D, NG), 0)
        out = selmean[...]
        cols = []
        for j in range(D):
            acc = selcov[j]
            for k in range(j):
                acc = acc - cols[k] * cols[k][j:j + 1, :]
            sq = jnp.sqrt(acc[j:j + 1, :])
            colj = jnp.where(isub >= j, acc / sq, jnp.float32(0.0))
            out = out + colj * z_ref[j:j + 1, :]
            cols.append(colj)
        out_ref[...] = out


_prep_call = pl.pallas_call(
    _prep_body,
    out_shape=[
        jax.ShapeDtypeStruct((1, NG), jnp.int32),
        jax.ShapeDtypeStruct((D, NG), jnp.float32),
    ],
    in_specs=[
        pl.BlockSpec(memory_space=pltpu.VMEM),
        pl.BlockSpec(memory_space=pltpu.SMEM),
    ],
    out_specs=[
        pl.BlockSpec(memory_space=pltpu.VMEM),
        pl.BlockSpec(memory_space=pltpu.VMEM),
    ],
)

_sweep_call = pl.pallas_call(
    _sweep_body,
    grid=(MC,),
    in_specs=[
        pl.BlockSpec((1, NG), lambda c: (0, 0)),
        pl.BlockSpec((D, NG), lambda c: (0, 0)),
        pl.BlockSpec((1, D, D, NG), lambda c: (c, 0, 0, 0)),
        pl.BlockSpec((1, D, NG), lambda c: (c, 0, 0)),
    ],
    out_specs=pl.BlockSpec((D, NG), lambda c: (0, 0)),
    out_shape=jax.ShapeDtypeStruct((D, NG), jnp.float32),
    scratch_shapes=[
        pltpu.VMEM((D, D, NG), jnp.float32),
        pltpu.VMEM((D, NG), jnp.float32),
    ],
)


def kernel(means, covs, weights, seed):
    wt = weights.T
    seed_arr = jnp.asarray(seed, jnp.int32).reshape(1, 1)
    comp2, z_t = _prep_call(wt, seed_arr)
    covt = jnp.transpose(covs, (1, 2, 3, 0))
    meant = jnp.transpose(means, (1, 2, 0))
    out_t = _sweep_call(comp2, z_t, covt, meant)
    return out_t.T
